# baseline (speedup 1.0000x reference)
import functools

import jax
import jax.numpy as jnp
from jax import lax
from jax.experimental import pallas as pl
from jax.experimental.pallas import tpu as pltpu
from jax.experimental.pallas import tpu_sc as plsc

_NN = 10000
_EE = 320000
_HH = 128
_AVG_D_LOG = 1.0

_NTILES = 32
_NPAD = 10240
_ROWS_PER_TILE = _NPAD // 16
_EDGE_W = 128
_ECP = 10112
_EPAD = _ECP * _NTILES
_NW = _ECP // _EDGE_W
_NB = 8


def _sc_sum_body(msgs_b, dst_b, msgs_c, dst_c, zeros_hbm, out,
                 msg_v, upd_v, idx_v, acc_sh):
    c = lax.axis_index("c")
    s = lax.axis_index("s")
    wid = c * 16 + s
    r0 = pl.multiple_of(s * _ROWS_PER_TILE, 8)
    base0 = wid * _ECP

    for si, (msgs_hbm, dst_hbm) in enumerate(((msgs_b, dst_b), (msgs_c, dst_c))):
        def round_body(r, carry0):
            b = 4 * c + r
            pltpu.sync_copy(zeros_hbm.at[pl.ds(r0, _ROWS_PER_TILE)],
                            acc_sh.at[pl.ds(r0, _ROWS_PER_TILE)])
            plsc.subcore_barrier()

            def window(i, carry):
                base = pl.multiple_of(base0 + i * _EDGE_W, 8)
                row8 = pl.multiple_of(base0 // 8 + i * (_EDGE_W // 8), 8)
                pltpu.sync_copy(dst_hbm.at[pl.ds(base, _EDGE_W)], idx_v)
                pltpu.sync_copy(
                    msgs_hbm.at[b, pl.ds(row8, _EDGE_W // 8)], msg_v)

                def row(j, carry2):
                    v = msg_v[j // 8, pl.ds((j % 8) * 16, 16)]
                    upd_v[j, pl.ds(0, 16)] = v
                    upd_v[j, pl.ds(16, 16)] = v * v
                    return carry2

                lax.fori_loop(0, _EDGE_W, row, 0, unroll=8)
                pltpu.sync_copy(upd_v, acc_sh.at[idx_v], add=True)
                return carry

            lax.fori_loop(0, _NW, window, 0)
            plsc.subcore_barrier()
            pltpu.sync_copy(acc_sh.at[pl.ds(r0, _ROWS_PER_TILE)],
                            out.at[si, b, pl.ds(r0, _ROWS_PER_TILE)])
            plsc.subcore_barrier()
            return carry0

        lax.fori_loop(0, 4, round_body, 0)


def _sc_sum_sumsq2(msgs_q_b, dst_b, msgs_q_c, dst_c, zeros32):
    mesh = plsc.VectorSubcoreMesh(core_axis_name="c", subcore_axis_name="s")
    f = pl.kernel(
        _sc_sum_body,
        out_type=pltpu.HBM((2, _NB, _NPAD, 32), jnp.float32),
        mesh=mesh,
        scratch_types=[
            pltpu.VMEM((_EDGE_W // 8, 128), jnp.float32),
            pltpu.VMEM((_EDGE_W, 32), jnp.float32),
            pltpu.VMEM((_EDGE_W,), jnp.int32),
            pltpu.VMEM_SHARED((_NPAD, 32), jnp.float32),
        ],
        compiler_params=pltpu.CompilerParams(use_tc_tiling_on_sc=False),
    )
    return f(msgs_q_b, dst_b, msgs_q_c, dst_c, zeros32)


def _mm_body(x_ref, w_ref, b_ref, o_ref, *, act):
    y = jnp.dot(x_ref[...], w_ref[...], preferred_element_type=jnp.float32)
    y = y + b_ref[...]
    if act == "relu":
        y = jax.nn.relu(y)
    o_ref[...] = y


def _pallas_mm(x, w, b, act="none", bm=1024):
    m, k = x.shape
    n = w.shape[1]
    mp = ((m + bm - 1) // bm) * bm
    if mp != m:
        x = jnp.pad(x, ((0, mp - m), (0, 0)))
    out = pl.pallas_call(
        functools.partial(_mm_body, act=act),
        grid=(mp // bm,),
        in_specs=[
            pl.BlockSpec((bm, k), lambda i: (i, 0)),
            pl.BlockSpec((k, n), lambda i: (0, 0)),
            pl.BlockSpec((n,), lambda i: (0,)),
        ],
        out_specs=pl.BlockSpec((bm, n), lambda i: (i, 0)),
        out_shape=jax.ShapeDtypeStruct((mp, n), jnp.float32),
    )(x, w, b)
    return out[:m] if mp != m else out


def _edge_gate_body(t_ref, w2_ref, b2_ref, wse_ref, bse_ref, o_ref):
    t = jax.nn.relu(t_ref[...])
    m = jnp.dot(t, w2_ref[...], preferred_element_type=jnp.float32) + b2_ref[...]
    gate = jax.nn.sigmoid(
        jnp.dot(m, wse_ref[...], preferred_element_type=jnp.float32) + bse_ref[...]
    )
    o_ref[...] = m * gate


def _edge_gate(t, w2, b2, wse, bse, bm=2048):
    m, k = t.shape
    n = w2.shape[1]
    mp = ((m + bm - 1) // bm) * bm
    if mp != m:
        t = jnp.pad(t, ((0, mp - m), (0, 0)))
    out = pl.pallas_call(
        _edge_gate_body,
        grid=(mp // bm,),
        in_specs=[
            pl.BlockSpec((bm, k), lambda i: (i, 0)),
            pl.BlockSpec((k, n), lambda i: (0, 0)),
            pl.BlockSpec((n,), lambda i: (0,)),
            pl.BlockSpec((n, 1), lambda i: (0, 0)),
            pl.BlockSpec((1,), lambda i: (0,)),
        ],
        out_specs=pl.BlockSpec((bm, n), lambda i: (i, 0)),
        out_shape=jax.ShapeDtypeStruct((mp, n), jnp.float32),
    )(t, w2, b2, wse, bse)
    return out[:m] if mp != m else out


def _pack_msgs(msgs):
    m = jnp.pad(msgs, ((0, _EPAD - _EE), (0, 0)))
    return m.reshape(_EPAD // 8, 8, _NB, 16).transpose(2, 0, 1, 3).reshape(
        _NB, _EPAD // 8, 128)


def _sum_sq_pad(msgs, dst, n):
    s = jax.ops.segment_sum(msgs, dst, num_segments=n)
    sq = jax.ops.segment_sum(msgs * msgs, dst, num_segments=n)
    s = jnp.pad(s, ((0, _NPAD - n), (0, 0)))
    sq = jnp.pad(sq, ((0, _NPAD - n), (0, 0)))
    return s, sq


def _maxmin_pad(msgs, dst, n):
    mx = jax.ops.segment_max(msgs, dst, num_segments=n)
    mn = jax.ops.segment_min(msgs, dst, num_segments=n)
    mx = jnp.pad(mx, ((0, _NPAD - n), (0, 0)))
    mn = jnp.pad(mn, ((0, _NPAD - n), (0, 0)))
    return mx, mn


def _degree(dst, n):
    ones = jnp.ones((dst.shape[0],), dtype=jnp.float32)
    deg = jax.ops.segment_sum(ones, dst, num_segments=n)
    return jnp.pad(deg, (0, _NPAD - n))[:, None]


def _stats_block(s_ref, sq_ref, mx_ref, mn_ref, deg_ref):
    deg = deg_ref[...]
    degc = jnp.maximum(deg, 1.0)
    mask = deg > 0
    logd = jnp.log(deg + 1.0)
    safe_logd = jnp.where(mask, logd, 1.0)
    s = s_ref[...]
    sq = sq_ref[...]
    mean = s / degc
    std = jnp.sqrt(jax.nn.relu(sq / degc - mean * mean) + 1e-5)
    mx = jnp.where(mask, mx_ref[...], 0.0)
    mn = jnp.where(mask, mn_ref[...], 0.0)
    h4 = jnp.concatenate([mean, mx, mn, std], axis=-1)
    f = jnp.concatenate(
        [h4, h4 * (logd / _AVG_D_LOG), h4 * (_AVG_D_LOG / safe_logd)], axis=-1)
    return jnp.where(mask, f, 0.0)


def _posttrans_body(h_ref, sb, sqb, mxb, mnb, degb, sc_, sqc, mxc, mnc, degc,
                    w_ref, b_ref, out_ref):
    h = h_ref[...]
    f_b = _stats_block(sb, sqb, mxb, mnb, degb)
    f_c = _stats_block(sc_, sqc, mxc, mnc, degc)
    xin = jnp.concatenate([h, f_b, f_c], axis=-1)
    out_ref[...] = (
        jnp.dot(xin, w_ref[...], preferred_element_type=jnp.float32)
        + b_ref[...] + h)


def _posttrans(h_pad, sb, sqb, mxb, mnb, degb, sc_, sqc, mxc, mnc, degc,
               wpost, bpost, bm=1024):
    nf_spec = pl.BlockSpec((bm, _HH), lambda i: (i, 0))
    d_spec = pl.BlockSpec((bm, 1), lambda i: (i, 0))
    return pl.pallas_call(
        _posttrans_body,
        grid=(_NPAD // bm,),
        in_specs=[
            nf_spec,
            nf_spec, nf_spec, nf_spec, nf_spec, d_spec,
            nf_spec, nf_spec, nf_spec, nf_spec, d_spec,
            pl.BlockSpec((25 * _HH, _HH), lambda i: (0, 0)),
            pl.BlockSpec((_HH,), lambda i: (0,)),
        ],
        out_specs=nf_spec,
        out_shape=jax.ShapeDtypeStruct((_NPAD, _HH), jnp.float32),
    )(h_pad, sb, sqb, mxb, mnb, degb, sc_, sqc, mxc, mnc, degc, wpost, bpost)


def kernel(x, edge_attr, params, edge_index_bond, edge_index_complete):
    n = x.shape[0]
    src, dst = edge_index_bond[0], edge_index_bond[1]
    srcc, dstc = edge_index_complete[0], edge_index_complete[1]

    zeros32 = jnp.zeros((_NPAD, 32), jnp.float32)
    x_pad = jnp.pad(x, ((0, _NPAD - n), (0, 0)))
    h0 = _pallas_mm(x_pad, params["node_in"][0][0], params["node_in"][0][1], act="relu")
    eh = _pallas_mm(edge_attr, params["edge_in"][0][0], params["edge_in"][0][1], act="relu")

    degb = _degree(dst, n)
    degc_ = _degree(dstc, n)
    dst_pad = jnp.pad(dst, (0, _EPAD - _EE))
    dstc_pad = jnp.pad(dstc, (0, _EPAD - _EE))

    def layer(h, p):
        wpre, bpre = p["pretrans"][0]
        ws, wd, we = wpre[:_HH], wpre[_HH : 2 * _HH], wpre[2 * _HH :]
        ab = _pallas_mm(h, jnp.concatenate([ws, wd], axis=1),
                        jnp.zeros((2 * _HH,), jnp.float32))
        a_tab, b_tab = ab[:, :_HH], ab[:, _HH:]
        c_edge = _pallas_mm(eh, we, bpre)
        e = a_tab[src] + b_tab[dst] + c_edge

        (w1, b1), (w2, b2) = p["pretrans_complete"]
        w1s, w1d = w1[:_HH], w1[_HH:]
        pq = _pallas_mm(h, jnp.concatenate([w1s, w1d], axis=1),
                        jnp.zeros((2 * _HH,), jnp.float32))
        p_tab, q_tab = pq[:, :_HH], pq[:, _HH:]
        t = p_tab[srcc] + q_tab[dstc] + b1
        wse, bse = p["soft_edge"]
        ec = _edge_gate(t, w2, b2, wse, bse)

        sb, sqb = _sum_sq_pad(e, dst, n)
        sc_, sqc = _sum_sq_pad(ec, dstc, n)
        mxb, mnb = _maxmin_pad(e, dst, n)
        mxc, mnc = _maxmin_pad(ec, dstc, n)
        wpost, bpost = p["posttrans"][0]
        h = _posttrans(h, sb, sqb, mxb, mnb, degb, sc_, sqc, mxc, mnc, degc_,
                       wpost, bpost)
        return h, 0

    stacked = jax.tree.map(lambda *xs: jnp.stack(xs), *params["layers"])
    h, _ = lax.scan(layer, h0, stacked)

    (wo1, bo1), (wo2, bo2) = params["node_out"]
    h = _pallas_mm(h, wo1, bo1, act="relu")
    h = _pallas_mm(h, wo2, bo2)
    h = h[:n]
    g = jnp.concatenate([h.sum(axis=0), h.mean(axis=0), h.max(axis=0)], axis=-1)

    (wr1, br1), (wr2, br2) = params["readout"]
    g = _pallas_mm(g[None, :], wr1, br1, act="relu", bm=8)
    g = _pallas_mm(g, wr2, br2, bm=8)
    return g[0]


# speedup vs baseline: 1.0545x; 1.0545x over previous
import functools

import jax
import jax.numpy as jnp
from jax import lax
from jax.experimental import pallas as pl
from jax.experimental.pallas import tpu as pltpu
from jax.experimental.pallas import tpu_sc as plsc

_NN = 10000
_EE = 320000
_HH = 128
_AVG_D_LOG = 1.0

_NTILES = 32
_NPAD = 10240
_ROWS_PER_TILE = _NPAD // 16
_EDGE_W = 128
_ECP = 10112
_EPAD = _ECP * _NTILES
_NW = _ECP // _EDGE_W
_NB = 8


def _sc_sum_body(msgs_b, dst_b, msgs_c, dst_c, zeros_hbm, out,
                 msg_v, upd_v, idx_v, acc_sh):
    c = lax.axis_index("c")
    s = lax.axis_index("s")
    wid = c * 16 + s
    r0 = pl.multiple_of(s * _ROWS_PER_TILE, 8)
    base0 = wid * _ECP

    for si, (msgs_hbm, dst_hbm) in enumerate(((msgs_b, dst_b), (msgs_c, dst_c))):
        def round_body(r, carry0):
            b = 4 * c + r
            pltpu.sync_copy(zeros_hbm.at[pl.ds(r0, _ROWS_PER_TILE)],
                            acc_sh.at[pl.ds(r0, _ROWS_PER_TILE)])
            plsc.subcore_barrier()

            def window(i, carry):
                base = pl.multiple_of(base0 + i * _EDGE_W, 8)
                row8 = pl.multiple_of(base0 // 8 + i * (_EDGE_W // 8), 8)
                pltpu.sync_copy(dst_hbm.at[pl.ds(base, _EDGE_W)], idx_v)
                pltpu.sync_copy(
                    msgs_hbm.at[b, pl.ds(row8, _EDGE_W // 8)], msg_v)

                def row(j, carry2):
                    v = msg_v[j // 8, pl.ds((j % 8) * 16, 16)]
                    upd_v[j, pl.ds(0, 16)] = v
                    upd_v[j, pl.ds(16, 16)] = v * v
                    return carry2

                lax.fori_loop(0, _EDGE_W, row, 0, unroll=8)
                pltpu.sync_copy(upd_v, acc_sh.at[idx_v], add=True)
                return carry

            lax.fori_loop(0, _NW, window, 0)
            plsc.subcore_barrier()
            pltpu.sync_copy(acc_sh.at[pl.ds(r0, _ROWS_PER_TILE)],
                            out.at[si, b, pl.ds(r0, _ROWS_PER_TILE)])
            plsc.subcore_barrier()
            return carry0

        lax.fori_loop(0, 4, round_body, 0)


def _sc_sum_sumsq2(msgs_q_b, dst_b, msgs_q_c, dst_c, zeros32):
    mesh = plsc.VectorSubcoreMesh(core_axis_name="c", subcore_axis_name="s")
    f = pl.kernel(
        _sc_sum_body,
        out_type=pltpu.HBM((2, _NB, _NPAD, 32), jnp.float32),
        mesh=mesh,
        scratch_types=[
            pltpu.VMEM((_EDGE_W // 8, 128), jnp.float32),
            pltpu.VMEM((_EDGE_W, 32), jnp.float32),
            pltpu.VMEM((_EDGE_W,), jnp.int32),
            pltpu.VMEM_SHARED((_NPAD, 32), jnp.float32),
        ],
        compiler_params=pltpu.CompilerParams(use_tc_tiling_on_sc=False),
    )
    return f(msgs_q_b, dst_b, msgs_q_c, dst_c, zeros32)


def _mm_body(x_ref, w_ref, b_ref, o_ref, *, act):
    y = jnp.dot(x_ref[...], w_ref[...], preferred_element_type=jnp.float32)
    y = y + b_ref[...]
    if act == "relu":
        y = jax.nn.relu(y)
    o_ref[...] = y


def _pallas_mm(x, w, b, act="none", bm=1024):
    m, k = x.shape
    n = w.shape[1]
    mp = ((m + bm - 1) // bm) * bm
    if mp != m:
        x = jnp.pad(x, ((0, mp - m), (0, 0)))
    out = pl.pallas_call(
        functools.partial(_mm_body, act=act),
        grid=(mp // bm,),
        in_specs=[
            pl.BlockSpec((bm, k), lambda i: (i, 0)),
            pl.BlockSpec((k, n), lambda i: (0, 0)),
            pl.BlockSpec((n,), lambda i: (0,)),
        ],
        out_specs=pl.BlockSpec((bm, n), lambda i: (i, 0)),
        out_shape=jax.ShapeDtypeStruct((mp, n), jnp.float32),
    )(x, w, b)
    return out[:m] if mp != m else out


def _edge_gate_body(t_ref, w2_ref, b2_ref, wse_ref, bse_ref, o_ref):
    t = jax.nn.relu(t_ref[...])
    m = jnp.dot(t, w2_ref[...], preferred_element_type=jnp.float32) + b2_ref[...]
    gate = jax.nn.sigmoid(
        jnp.dot(m, wse_ref[...], preferred_element_type=jnp.float32) + bse_ref[...]
    )
    o_ref[...] = m * gate


def _edge_gate(t, w2, b2, wse, bse, bm=2048):
    m, k = t.shape
    n = w2.shape[1]
    mp = ((m + bm - 1) // bm) * bm
    if mp != m:
        t = jnp.pad(t, ((0, mp - m), (0, 0)))
    out = pl.pallas_call(
        _edge_gate_body,
        grid=(mp // bm,),
        in_specs=[
            pl.BlockSpec((bm, k), lambda i: (i, 0)),
            pl.BlockSpec((k, n), lambda i: (0, 0)),
            pl.BlockSpec((n,), lambda i: (0,)),
            pl.BlockSpec((n, 1), lambda i: (0, 0)),
            pl.BlockSpec((1,), lambda i: (0,)),
        ],
        out_specs=pl.BlockSpec((bm, n), lambda i: (i, 0)),
        out_shape=jax.ShapeDtypeStruct((mp, n), jnp.float32),
    )(t, w2, b2, wse, bse)
    return out[:m] if mp != m else out


def _pack_msgs(msgs):
    m = jnp.pad(msgs, ((0, _EPAD - _EE), (0, 0)))
    return m.reshape(_EPAD // 8, 8, _NB, 16).transpose(2, 0, 1, 3).reshape(
        _NB, _EPAD // 8, 128)


def _sum_sq_pad(msgs, dst, n):
    s = jax.ops.segment_sum(msgs, dst, num_segments=n)
    sq = jax.ops.segment_sum(msgs * msgs, dst, num_segments=n)
    s = jnp.pad(s, ((0, _NPAD - n), (0, 0)))
    sq = jnp.pad(sq, ((0, _NPAD - n), (0, 0)))
    return s, sq


def _maxmin_pad(msgs, dst, n):
    mx = jax.ops.segment_max(msgs, dst, num_segments=n)
    mn = jax.ops.segment_min(msgs, dst, num_segments=n)
    mx = jnp.pad(mx, ((0, _NPAD - n), (0, 0)))
    mn = jnp.pad(mn, ((0, _NPAD - n), (0, 0)))
    return mx, mn


def _degree(dst, n):
    ones = jnp.ones((dst.shape[0],), dtype=jnp.float32)
    deg = jax.ops.segment_sum(ones, dst, num_segments=n)
    return jnp.pad(deg, (0, _NPAD - n))[:, None]


def _stats_block(s_ref, sq_ref, mx_ref, mn_ref, deg_ref):
    deg = deg_ref[...]
    degc = jnp.maximum(deg, 1.0)
    mask = deg > 0
    logd = jnp.log(deg + 1.0)
    safe_logd = jnp.where(mask, logd, 1.0)
    s = s_ref[...]
    sq = sq_ref[...]
    mean = s / degc
    std = jnp.sqrt(jax.nn.relu(sq / degc - mean * mean) + 1e-5)
    mx = jnp.where(mask, mx_ref[...], 0.0)
    mn = jnp.where(mask, mn_ref[...], 0.0)
    h4 = jnp.concatenate([mean, mx, mn, std], axis=-1)
    f = jnp.concatenate(
        [h4, h4 * (logd / _AVG_D_LOG), h4 * (_AVG_D_LOG / safe_logd)], axis=-1)
    return jnp.where(mask, f, 0.0)


def _posttrans_body(h_ref, sb, sqb, mxb, mnb, degb, sc_, sqc, mxc, mnc, degc,
                    w_ref, b_ref, out_ref):
    h = h_ref[...]
    f_b = _stats_block(sb, sqb, mxb, mnb, degb)
    f_c = _stats_block(sc_, sqc, mxc, mnc, degc)
    xin = jnp.concatenate([h, f_b, f_c], axis=-1)
    out_ref[...] = (
        jnp.dot(xin, w_ref[...], preferred_element_type=jnp.float32)
        + b_ref[...] + h)


def _posttrans(h_pad, sb, sqb, mxb, mnb, degb, sc_, sqc, mxc, mnc, degc,
               wpost, bpost, bm=1024):
    nf_spec = pl.BlockSpec((bm, _HH), lambda i: (i, 0))
    d_spec = pl.BlockSpec((bm, 1), lambda i: (i, 0))
    return pl.pallas_call(
        _posttrans_body,
        grid=(_NPAD // bm,),
        in_specs=[
            nf_spec,
            nf_spec, nf_spec, nf_spec, nf_spec, d_spec,
            nf_spec, nf_spec, nf_spec, nf_spec, d_spec,
            pl.BlockSpec((25 * _HH, _HH), lambda i: (0, 0)),
            pl.BlockSpec((_HH,), lambda i: (0,)),
        ],
        out_specs=nf_spec,
        out_shape=jax.ShapeDtypeStruct((_NPAD, _HH), jnp.float32),
    )(h_pad, sb, sqb, mxb, mnb, degb, sc_, sqc, mxc, mnc, degc, wpost, bpost)


def kernel(x, edge_attr, params, edge_index_bond, edge_index_complete):
    n = x.shape[0]
    src, dst = edge_index_bond[0], edge_index_bond[1]
    srcc, dstc = edge_index_complete[0], edge_index_complete[1]

    zeros32 = jnp.zeros((_NPAD, 32), jnp.float32)
    x_pad = jnp.pad(x, ((0, _NPAD - n), (0, 0)))
    h0 = _pallas_mm(x_pad, params["node_in"][0][0], params["node_in"][0][1], act="relu")
    eh = _pallas_mm(edge_attr, params["edge_in"][0][0], params["edge_in"][0][1], act="relu")

    degb = _degree(dst, n)
    degc_ = _degree(dstc, n)
    dst_pad = jnp.pad(dst, (0, _EPAD - _EE))
    dstc_pad = jnp.pad(dstc, (0, _EPAD - _EE))

    def layer(h, p):
        wpre, bpre = p["pretrans"][0]
        ws, wd, we = wpre[:_HH], wpre[_HH : 2 * _HH], wpre[2 * _HH :]
        ab = _pallas_mm(h, jnp.concatenate([ws, wd], axis=1),
                        jnp.zeros((2 * _HH,), jnp.float32))
        a_tab, b_tab = ab[:, :_HH], ab[:, _HH:]
        c_edge = _pallas_mm(eh, we, bpre)
        e = a_tab[src] + b_tab[dst] + c_edge

        (w1, b1), (w2, b2) = p["pretrans_complete"]
        w1s, w1d = w1[:_HH], w1[_HH:]
        pq = _pallas_mm(h, jnp.concatenate([w1s, w1d], axis=1),
                        jnp.zeros((2 * _HH,), jnp.float32))
        p_tab, q_tab = pq[:, :_HH], pq[:, _HH:]
        t = p_tab[srcc] + q_tab[dstc] + b1
        wse, bse = p["soft_edge"]
        ec = _edge_gate(t, w2, b2, wse, bse)

        sb, sqb = _sum_sq_pad(e, dst, n)
        sc_, sqc = _sum_sq_pad(ec, dstc, n)
        mxb, mnb = _maxmin_pad(e, dst, n)
        mxc, mnc = _maxmin_pad(ec, dstc, n)
        wpost, bpost = p["posttrans"][0]
        h = _posttrans(h, sb, sqb, mxb, mnb, degb, sc_, sqc, mxc, mnc, degc_,
                       wpost, bpost)
        return h, 0

    h = h0
    for p in params["layers"]:
        h, _ = layer(h, p)

    (wo1, bo1), (wo2, bo2) = params["node_out"]
    h = _pallas_mm(h, wo1, bo1, act="relu")
    h = _pallas_mm(h, wo2, bo2)
    h = h[:n]
    g = jnp.concatenate([h.sum(axis=0), h.mean(axis=0), h.max(axis=0)], axis=-1)

    (wr1, br1), (wr2, br2) = params["readout"]
    g = _pallas_mm(g[None, :], wr1, br1, act="relu", bm=8)
    g = _pallas_mm(g, wr2, br2, bm=8)
    return g[0]
